# baseline (speedup 1.0000x reference)
import functools

import jax
import jax.numpy as jnp
from jax import lax
from jax.experimental import pallas as pl
from jax.experimental.pallas import tpu as pltpu
from jax.experimental.pallas import tpu_sc as plsc

L = 16
NC = 2
NS = 16
NW = NC * NS
ROWS = 128
COLS = 32768
RPW = ROWS // NW
CHUNKS = COLS // L
NSEG = 16
SEGC = CHUNKS // NSEG
IMAX = 2**31 - 1


def _lane_shuffle(v, perm):
    return lax.gather(
        v, perm[:, None],
        lax.GatherDimensionNumbers(
            offset_dims=(), collapsed_slice_dims=(0,), start_index_map=(0,)),
        slice_sizes=(1,),
        mode=lax.GatherScatterMode.PROMISE_IN_BOUNDS)


def _bfly_max(v, iota):
    for k in (1, 2, 4, 8):
        v = jnp.maximum(v, _lane_shuffle(v, iota ^ k))
    return v


def _bfly_min(v, iota):
    for k in (1, 2, 4, 8):
        v = jnp.minimum(v, _lane_shuffle(v, iota ^ k))
    return v


def _row_argmax(buf, iota, tmp):
    ninf = jnp.full((L,), -jnp.inf, dtype=jnp.float32)
    imaxv = jnp.full((L,), IMAX, dtype=jnp.int32)

    def p1body(i, accs):
        accs = list(accs)
        for g in range(NSEG):
            vals = buf[pl.ds((g * SEGC + i) * L, L)]
            accs[g] = jnp.maximum(accs[g], vals)
        return tuple(accs)

    accs = lax.fori_loop(0, SEGC, p1body, (ninf,) * NSEG, unroll=2)

    t = list(accs)
    while len(t) > 1:
        t = [jnp.maximum(t[2 * j], t[2 * j + 1]) for j in range(len(t) // 2)]
    gmax = _bfly_max(t[0], iota)

    segv = imaxv
    for g in reversed(range(NSEG)):
        segv = jnp.where(accs[g] == gmax, jnp.int32(g), segv)
    segv = _bfly_min(segv, iota)
    base = segv[0] * SEGC

    def p2body(i, run):
        c = base + i
        vals = buf[pl.ds(c * L, L)]
        idxv = lax.broadcast(c * L, (L,)) + iota
        cand = jnp.where(vals == gmax, idxv, imaxv)
        return jnp.minimum(run, cand)

    run = lax.fori_loop(0, SEGC, p2body, imaxv, unroll=2)
    return _bfly_min(run, iota)


def _argmax_kernel_body(x_hbm, out_hbm, buf0, buf1, outv, tmp, sem0, sem1):
    wid = lax.axis_index("s") * NC + lax.axis_index("c")
    base = wid * RPW
    bufs = [buf0, buf1]
    sems = [sem0, sem1]
    iota = lax.iota(jnp.int32, L)

    cps = [None] * RPW
    cps[0] = pltpu.async_copy(x_hbm.at[base], bufs[0], sems[0])
    res = jnp.zeros((L,), dtype=jnp.int32)
    for r in range(RPW):
        if r + 1 < RPW:
            cps[r + 1] = pltpu.async_copy(
                x_hbm.at[base + r + 1], bufs[(r + 1) % 2], sems[(r + 1) % 2])
        cps[r].wait()
        ans = _row_argmax(bufs[r % 2], iota, tmp)
        res = jnp.where(iota == r, ans, res)
    outv[...] = res
    pltpu.sync_copy(outv, out_hbm.at[wid])


@jax.jit
def kernel(x):
    mesh = plsc.VectorSubcoreMesh(core_axis_name="c", subcore_axis_name="s")
    out = pl.kernel(
        _argmax_kernel_body,
        mesh=mesh,
        out_type=jax.ShapeDtypeStruct((NW, L), jnp.int32),
        scratch_types=[
            pltpu.VMEM((COLS,), jnp.float32),
            pltpu.VMEM((COLS,), jnp.float32),
            pltpu.VMEM((L,), jnp.int32),
            pltpu.VMEM((L,), jnp.int32),
            pltpu.SemaphoreType.DMA,
            pltpu.SemaphoreType.DMA,
        ],
    )(x)
    return out[:, :RPW].reshape(ROWS).astype(jnp.int64)


# speedup vs baseline: 2.7556x; 2.7556x over previous
import jax
import jax.numpy as jnp
from jax import lax
from jax.experimental import pallas as pl
from jax.experimental.pallas import tpu as pltpu

ROWS = 128
COLS = 32768
BC = 4096
NB = COLS // BC
LANES = 128
IMAX = 2**31 - 1


def _tc_body(x_ref, out_ref, accv_ref, acci_ref):
    j = pl.program_id(0)

    @pl.when(j == 0)
    def _init():
        accv_ref[...] = jnp.full((ROWS, LANES), -jnp.inf, dtype=jnp.float32)
        acci_ref[...] = jnp.zeros((ROWS, LANES), dtype=jnp.int32)

    blk = x_ref[...]
    accv = accv_ref[...]
    acci = acci_ref[...]
    lane = lax.broadcasted_iota(jnp.int32, (ROWS, LANES), 1)
    for k in range(BC // LANES):
        sub = blk[:, k * LANES:(k + 1) * LANES]
        pred = sub > accv
        accv = jnp.where(pred, sub, accv)
        acci = jnp.where(pred, lane + (j * BC + k * LANES), acci)
    accv_ref[...] = accv
    acci_ref[...] = acci

    @pl.when(j == NB - 1)
    def _finish():
        gmax = jnp.max(accv, axis=1, keepdims=True)
        cand = jnp.where(accv == gmax, acci,
                         jnp.full((ROWS, LANES), IMAX, dtype=jnp.int32))
        res = jnp.min(cand, axis=1, keepdims=True)
        out_ref[...] = jnp.broadcast_to(res, (ROWS, LANES))


@jax.jit
def kernel(x):
    out = pl.pallas_call(
        _tc_body,
        grid=(NB,),
        in_specs=[pl.BlockSpec((ROWS, BC), lambda j: (0, j))],
        out_specs=pl.BlockSpec((ROWS, LANES), lambda j: (0, 0)),
        out_shape=jax.ShapeDtypeStruct((ROWS, LANES), jnp.int32),
        scratch_shapes=[
            pltpu.VMEM((ROWS, LANES), jnp.float32),
            pltpu.VMEM((ROWS, LANES), jnp.int32),
        ],
    )(x)
    return out[:, 0].astype(jnp.int64)
